# Initial kernel scaffold; baseline (speedup 1.0000x reference)
import jax
import jax.numpy as jnp
from jax.experimental import pallas as pl


def kernel(x, edge_index, edge_weight, W1, b1, W2, b2, W3, b3, W4, b4):
    raise NotImplementedError("write your pallas kernel here")


# speedup vs baseline: 7.6136x; 7.6136x over previous
import functools
import jax
import jax.numpy as jnp
from jax import lax
from jax.experimental import pallas as pl
from jax.experimental.pallas import tpu as pltpu
from jax.experimental.pallas import tpu_sc as plsc

N = 10000
E = 320000
D = 128

NC = 2
NS = 16
NW = NC * NS
EPT = E // NW
CHUNK = 80
NCHUNK = EPT // CHUNK
NP = 10240
RPT = NP // NS
ZR = RPT // 5
DEGP = 10240
DPT = DEGP // NS


@functools.cache
def _build_sc_degree():
  mesh = plsc.VectorSubcoreMesh(core_axis_name="c", subcore_axis_name="s")

  @functools.partial(
      pl.kernel,
      out_type=jax.ShapeDtypeStruct((NC, DEGP), jnp.float32),
      mesh=mesh,
      scratch_types=[
          pltpu.MemorySpace.VMEM_SHARED((DEGP,), jnp.float32),
          pltpu.MemorySpace.VMEM((DPT,), jnp.float32),
          pltpu.MemorySpace.VMEM((CHUNK,), jnp.int32),
          pltpu.MemorySpace.VMEM((CHUNK,), jnp.float32),
      ],
  )
  def sc_degree(dst_hbm, ew_hbm, deg_out, dacc, zv, dst_v, ew_v):
    c = lax.axis_index("c")
    s = lax.axis_index("s")
    wid = c * NS + s
    for j in range(DPT // 16):
      zv[pl.ds(j * 16, 16)] = jnp.zeros((16,), jnp.float32)
    pltpu.sync_copy(zv, dacc.at[pl.ds(s * DPT, DPT)])
    plsc.subcore_barrier()

    base = wid * EPT

    def body(i, carry):
      off = base + i * CHUNK
      pltpu.sync_copy(dst_hbm.at[pl.ds(off, CHUNK)], dst_v)
      pltpu.sync_copy(ew_hbm.at[pl.ds(off, CHUNK)], ew_v)
      pltpu.sync_copy(ew_v, dacc.at[dst_v], add=True)
      return carry

    lax.fori_loop(0, NCHUNK, body, 0)
    plsc.subcore_barrier()
    pltpu.sync_copy(dacc.at[pl.ds(s * DPT, DPT)],
                    deg_out.at[c, pl.ds(s * DPT, DPT)])

  return sc_degree


@functools.cache
def _build_sc_agg():
  mesh = plsc.VectorSubcoreMesh(core_axis_name="c", subcore_axis_name="s")

  @functools.partial(
      pl.kernel,
      out_type=jax.ShapeDtypeStruct((NC, NP, D), jnp.float32),
      mesh=mesh,
      scratch_types=[
          pltpu.MemorySpace.VMEM_SHARED((NP, D), jnp.float32),
          pltpu.MemorySpace.VMEM((ZR, D), jnp.float32),
          pltpu.MemorySpace.VMEM((CHUNK,), jnp.int32),
          pltpu.MemorySpace.VMEM((CHUNK,), jnp.int32),
          pltpu.MemorySpace.VMEM((CHUNK,), jnp.float32),
          pltpu.MemorySpace.VMEM((CHUNK, D), jnp.float32),
      ],
  )
  def sc_agg(g_hbm, src_hbm, dst_hbm, ew_hbm, part_out,
             acc, zbuf, src_v, dst_v, ew_v, rows):
    c = lax.axis_index("c")
    s = lax.axis_index("s")
    wid = c * NS + s

    def zrow(r, carry):
      for f in range(D // 16):
        zbuf[r, pl.ds(f * 16, 16)] = jnp.zeros((16,), jnp.float32)
      return carry

    lax.fori_loop(0, ZR, zrow, 0)
    for j in range(RPT // ZR):
      pltpu.sync_copy(zbuf, acc.at[pl.ds(s * RPT + j * ZR, ZR)])
    plsc.subcore_barrier()

    base = wid * EPT

    def body(i, carry):
      off = base + i * CHUNK
      pltpu.sync_copy(src_hbm.at[pl.ds(off, CHUNK)], src_v)
      pltpu.sync_copy(dst_hbm.at[pl.ds(off, CHUNK)], dst_v)
      pltpu.sync_copy(ew_hbm.at[pl.ds(off, CHUNK)], ew_v)
      pltpu.sync_copy(g_hbm.at[src_v], rows)

      def scale(j, c2):
        ew16 = ew_v[pl.ds(j * 16, 16)]
        for k in range(16):
          w = ew16[k]
          e = j * 16 + k
          for f in range(D // 16):
            rows[e, pl.ds(f * 16, 16)] = rows[e, pl.ds(f * 16, 16)] * w
        return c2

      lax.fori_loop(0, CHUNK // 16, scale, 0)
      pltpu.sync_copy(rows, acc.at[dst_v], add=True)
      return carry

    lax.fori_loop(0, NCHUNK, body, 0)
    plsc.subcore_barrier()
    pltpu.sync_copy(acc.at[pl.ds(s * RPT, RPT)],
                    part_out.at[c, pl.ds(s * RPT, RPT)])

  return sc_agg


def _dinv_body(dp_ref, out_ref):
  deg = dp_ref[0] + dp_ref[1] + 1.0
  out_ref[...] = jnp.where(deg > 0, 1.0 / jnp.sqrt(deg), 0.0)


def _tc_dinv(deg_pair):
  dp = deg_pair.reshape(NC, DEGP // D, D)
  out = pl.pallas_call(
      _dinv_body,
      out_shape=jax.ShapeDtypeStruct((DEGP // D, D), jnp.float32),
  )(dp)
  return out.reshape(DEGP, 1)[:N]


BLK = 1000
GRID = N // BLK


def _pre_body(x_ref, w_ref, dinv_ref, g_ref):
  h = jnp.dot(x_ref[...], w_ref[...], preferred_element_type=jnp.float32)
  g_ref[...] = h * dinv_ref[...]


def _tc_pre(x, W, dinv):
  return pl.pallas_call(
      _pre_body,
      grid=(GRID,),
      in_specs=[
          pl.BlockSpec((BLK, D), lambda i: (i, 0)),
          pl.BlockSpec((D, D), lambda i: (0, 0)),
          pl.BlockSpec((BLK, 1), lambda i: (i, 0)),
      ],
      out_specs=pl.BlockSpec((BLK, D), lambda i: (i, 0)),
      out_shape=jax.ShapeDtypeStruct((N, D), jnp.float32),
  )(x, W, dinv)


def _mid_body(p0_ref, p1_ref, g_ref, dinv_ref, b_ref, w_ref, out_ref):
  a = jax.nn.relu(dinv_ref[...] * (p0_ref[...] + p1_ref[...] + g_ref[...])
                  + b_ref[...])
  h = jnp.dot(a, w_ref[...], preferred_element_type=jnp.float32)
  out_ref[...] = h * dinv_ref[...]


def _tc_mid(part, g, dinv, b, Wn):
  return pl.pallas_call(
      _mid_body,
      grid=(GRID,),
      in_specs=[
          pl.BlockSpec((BLK, D), lambda i: (i, 0)),
          pl.BlockSpec((BLK, D), lambda i: (i, 0)),
          pl.BlockSpec((BLK, D), lambda i: (i, 0)),
          pl.BlockSpec((BLK, 1), lambda i: (i, 0)),
          pl.BlockSpec((1, D), lambda i: (0, 0)),
          pl.BlockSpec((D, D), lambda i: (0, 0)),
      ],
      out_specs=pl.BlockSpec((BLK, D), lambda i: (i, 0)),
      out_shape=jax.ShapeDtypeStruct((N, D), jnp.float32),
  )(part[0], part[1], g, dinv, b.reshape(1, D), Wn)


def _fin_body(p0_ref, p1_ref, g_ref, dinv_ref, b_ref, out_ref):
  out_ref[...] = jax.nn.sigmoid(
      dinv_ref[...] * (p0_ref[...] + p1_ref[...] + g_ref[...]) + b_ref[...])


def _tc_fin(part, g, dinv, b):
  return pl.pallas_call(
      _fin_body,
      grid=(GRID,),
      in_specs=[
          pl.BlockSpec((BLK, D), lambda i: (i, 0)),
          pl.BlockSpec((BLK, D), lambda i: (i, 0)),
          pl.BlockSpec((BLK, D), lambda i: (i, 0)),
          pl.BlockSpec((BLK, 1), lambda i: (i, 0)),
          pl.BlockSpec((1, D), lambda i: (0, 0)),
      ],
      out_specs=pl.BlockSpec((BLK, D), lambda i: (i, 0)),
      out_shape=jax.ShapeDtypeStruct((N, D), jnp.float32),
  )(part[0], part[1], g, dinv, b.reshape(1, D))


@jax.jit
def kernel(x, edge_index, edge_weight, W1, b1, W2, b2, W3, b3, W4, b4):
  src = edge_index[0]
  dst = edge_index[1]

  deg_pair = _build_sc_degree()(dst, edge_weight)
  dinv = _tc_dinv(deg_pair)

  sc_agg = _build_sc_agg()
  g = _tc_pre(x, W1, dinv)
  part = sc_agg(g, src, dst, edge_weight)
  g = _tc_mid(part, g, dinv, b1, W2)
  part = sc_agg(g, src, dst, edge_weight)
  g = _tc_mid(part, g, dinv, b2, W3)
  part = sc_agg(g, src, dst, edge_weight)
  g = _tc_mid(part, g, dinv, b3, W4)
  part = sc_agg(g, src, dst, edge_weight)
  return _tc_fin(part, g, dinv, b4)


# speedup vs baseline: 17.6596x; 2.3195x over previous
import functools
import jax
import jax.numpy as jnp
from jax import lax
from jax.experimental import pallas as pl
from jax.experimental.pallas import tpu as pltpu
from jax.experimental.pallas import tpu_sc as plsc

N = 10000
E = 320000
D = 128

NC = 2
NS = 16
NW = NC * NS
EPT = E // NW
CHUNK = 80
NCHUNK = EPT // CHUNK
NP = 10240
RPT = NP // NS
ZR = RPT // 5
DEGP = 10240
DPT = DEGP // NS


@functools.cache
def _build_sc_degree():
  mesh = plsc.VectorSubcoreMesh(core_axis_name="c", subcore_axis_name="s")

  @functools.partial(
      pl.kernel,
      out_type=jax.ShapeDtypeStruct((NC, DEGP), jnp.float32),
      mesh=mesh,
      scratch_types=[
          pltpu.MemorySpace.VMEM_SHARED((DEGP,), jnp.float32),
          pltpu.MemorySpace.VMEM((DPT,), jnp.float32),
          pltpu.MemorySpace.VMEM((CHUNK,), jnp.int32),
          pltpu.MemorySpace.VMEM((CHUNK,), jnp.float32),
      ],
  )
  def sc_degree(dst_hbm, ew_hbm, deg_out, dacc, zv, dst_v, ew_v):
    c = lax.axis_index("c")
    s = lax.axis_index("s")
    wid = c * NS + s
    for j in range(DPT // 16):
      zv[pl.ds(j * 16, 16)] = jnp.zeros((16,), jnp.float32)
    pltpu.sync_copy(zv, dacc.at[pl.ds(s * DPT, DPT)])
    plsc.subcore_barrier()

    base = wid * EPT

    def body(i, carry):
      off = base + i * CHUNK
      pltpu.sync_copy(dst_hbm.at[pl.ds(off, CHUNK)], dst_v)
      pltpu.sync_copy(ew_hbm.at[pl.ds(off, CHUNK)], ew_v)
      pltpu.sync_copy(ew_v, dacc.at[dst_v], add=True)
      return carry

    lax.fori_loop(0, NCHUNK, body, 0)
    plsc.subcore_barrier()
    pltpu.sync_copy(dacc.at[pl.ds(s * DPT, DPT)],
                    deg_out.at[c, pl.ds(s * DPT, DPT)])

  return sc_degree


@functools.cache
def _build_sc_agg():
  mesh = plsc.VectorSubcoreMesh(core_axis_name="c", subcore_axis_name="s")

  @functools.partial(
      pl.kernel,
      out_type=jax.ShapeDtypeStruct((NC, NP, D), jnp.float32),
      mesh=mesh,
      scratch_types=[
          pltpu.MemorySpace.VMEM_SHARED((NP, D), jnp.float32),
          pltpu.MemorySpace.VMEM((CHUNK, D), jnp.float32),
          pltpu.MemorySpace.VMEM((CHUNK, D), jnp.float32),
          [pltpu.MemorySpace.VMEM((CHUNK,), jnp.int32) for _ in range(4)],
          [pltpu.MemorySpace.VMEM((CHUNK,), jnp.int32) for _ in range(4)],
          [pltpu.MemorySpace.VMEM((CHUNK,), jnp.float32) for _ in range(4)],
          pltpu.SemaphoreType.DMA,
          pltpu.SemaphoreType.DMA,
          [pltpu.SemaphoreType.DMA for _ in range(4)],
      ],
  )
  def sc_agg(g_hbm, src_hbm, dst_hbm, ew_hbm, part_out,
             acc, rows0, rows1, src_v, dst_v, ew_v, gsem0, gsem1, isem):
    c = lax.axis_index("c")
    s = lax.axis_index("s")
    wid = c * NS + s
    rows = (rows0, rows1)
    gsem = (gsem0, gsem1)
    base = wid * EPT

    def ifetch(j, k):
      off = base + j * CHUNK
      pltpu.async_copy(src_hbm.at[pl.ds(off, CHUNK)], src_v[k], isem[k])
      pltpu.async_copy(dst_hbm.at[pl.ds(off, CHUNK)], dst_v[k], isem[k])
      pltpu.async_copy(ew_hbm.at[pl.ds(off, CHUNK)], ew_v[k], isem[k])

    def iwait(k):
      pltpu.make_async_copy(src_hbm.at[pl.ds(0, CHUNK)], src_v[k], isem[k]).wait()
      pltpu.make_async_copy(dst_hbm.at[pl.ds(0, CHUNK)], dst_v[k], isem[k]).wait()
      pltpu.make_async_copy(ew_hbm.at[pl.ds(0, CHUNK)], ew_v[k], isem[k]).wait()

    def gather(k, b):
      pltpu.async_copy(g_hbm.at[src_v[k]], rows[b], gsem[b])

    def gwait(b):
      pltpu.make_async_copy(g_hbm.at[src_v[0]], rows[b], gsem[b]).wait()

    for k in range(4):
      ifetch(k, k)

    def zrow(r, carry):
      for f in range(D // 16):
        rows0[r, pl.ds(f * 16, 16)] = jnp.zeros((16,), jnp.float32)
      return carry

    lax.fori_loop(0, CHUNK, zrow, 0)
    for j in range(RPT // CHUNK):
      pltpu.sync_copy(rows0, acc.at[pl.ds(s * RPT + j * CHUNK, CHUNK)])
    plsc.subcore_barrier()

    iwait(0)
    gather(0, 0)
    iwait(1)
    gather(1, 1)

    def process(k, b):
      r = rows[b]

      def scale(m, c2):
        ew16 = ew_v[k][pl.ds(m * 16, 16)]
        for t in range(16):
          w = ew16[t]
          e = m * 16 + t
          for f in range(D // 16):
            r[e, pl.ds(f * 16, 16)] = r[e, pl.ds(f * 16, 16)] * w
        return c2

      lax.fori_loop(0, CHUNK // 16, scale, 0)
      pltpu.sync_copy(r, acc.at[dst_v[k]], add=True)

    def do_slot(j, k, fetch, issue):
      b = k % 2
      gwait(b)
      process(k, b)
      if fetch:
        ifetch(j + 4, k)
      if issue:
        iwait((k + 2) % 4)
        gather((k + 2) % 4, b)

    def body(q, carry):
      j0 = 4 * q
      for k in range(4):
        do_slot(j0 + k, k, True, True)
      return carry

    lax.fori_loop(0, 30, body, 0)
    do_slot(120, 0, True, True)
    do_slot(121, 1, False, True)
    do_slot(122, 2, False, True)
    do_slot(123, 3, False, False)
    do_slot(124, 0, False, False)

    plsc.subcore_barrier()
    pltpu.sync_copy(acc.at[pl.ds(s * RPT, RPT)],
                    part_out.at[c, pl.ds(s * RPT, RPT)])

  return sc_agg


def _dinv_body(dp_ref, out_ref):
  deg = dp_ref[0] + dp_ref[1] + 1.0
  out_ref[...] = jnp.where(deg > 0, 1.0 / jnp.sqrt(deg), 0.0)


def _tc_dinv(deg_pair):
  dp = deg_pair.reshape(NC, DEGP // D, D)
  out = pl.pallas_call(
      _dinv_body,
      out_shape=jax.ShapeDtypeStruct((DEGP // D, D), jnp.float32),
  )(dp)
  return out.reshape(DEGP, 1)[:N]


BLK = 1000
GRID = N // BLK


def _pre_body(x_ref, w_ref, dinv_ref, g_ref):
  h = jnp.dot(x_ref[...], w_ref[...], preferred_element_type=jnp.float32)
  g_ref[...] = h * dinv_ref[...]


def _tc_pre(x, W, dinv):
  return pl.pallas_call(
      _pre_body,
      grid=(GRID,),
      in_specs=[
          pl.BlockSpec((BLK, D), lambda i: (i, 0)),
          pl.BlockSpec((D, D), lambda i: (0, 0)),
          pl.BlockSpec((BLK, 1), lambda i: (i, 0)),
      ],
      out_specs=pl.BlockSpec((BLK, D), lambda i: (i, 0)),
      out_shape=jax.ShapeDtypeStruct((N, D), jnp.float32),
  )(x, W, dinv)


def _mid_body(p0_ref, p1_ref, g_ref, dinv_ref, b_ref, w_ref, out_ref):
  a = jax.nn.relu(dinv_ref[...] * (p0_ref[...] + p1_ref[...] + g_ref[...])
                  + b_ref[...])
  h = jnp.dot(a, w_ref[...], preferred_element_type=jnp.float32)
  out_ref[...] = h * dinv_ref[...]


def _tc_mid(part, g, dinv, b, Wn):
  return pl.pallas_call(
      _mid_body,
      grid=(GRID,),
      in_specs=[
          pl.BlockSpec((BLK, D), lambda i: (i, 0)),
          pl.BlockSpec((BLK, D), lambda i: (i, 0)),
          pl.BlockSpec((BLK, D), lambda i: (i, 0)),
          pl.BlockSpec((BLK, 1), lambda i: (i, 0)),
          pl.BlockSpec((1, D), lambda i: (0, 0)),
          pl.BlockSpec((D, D), lambda i: (0, 0)),
      ],
      out_specs=pl.BlockSpec((BLK, D), lambda i: (i, 0)),
      out_shape=jax.ShapeDtypeStruct((N, D), jnp.float32),
  )(part[0], part[1], g, dinv, b.reshape(1, D), Wn)


def _fin_body(p0_ref, p1_ref, g_ref, dinv_ref, b_ref, out_ref):
  out_ref[...] = jax.nn.sigmoid(
      dinv_ref[...] * (p0_ref[...] + p1_ref[...] + g_ref[...]) + b_ref[...])


def _tc_fin(part, g, dinv, b):
  return pl.pallas_call(
      _fin_body,
      grid=(GRID,),
      in_specs=[
          pl.BlockSpec((BLK, D), lambda i: (i, 0)),
          pl.BlockSpec((BLK, D), lambda i: (i, 0)),
          pl.BlockSpec((BLK, D), lambda i: (i, 0)),
          pl.BlockSpec((BLK, 1), lambda i: (i, 0)),
          pl.BlockSpec((1, D), lambda i: (0, 0)),
      ],
      out_specs=pl.BlockSpec((BLK, D), lambda i: (i, 0)),
      out_shape=jax.ShapeDtypeStruct((N, D), jnp.float32),
  )(part[0], part[1], g, dinv, b.reshape(1, D))


@jax.jit
def kernel(x, edge_index, edge_weight, W1, b1, W2, b2, W3, b3, W4, b4):
  src = edge_index[0]
  dst = edge_index[1]


  deg_pair = _build_sc_degree()(dst, edge_weight)
  dinv = _tc_dinv(deg_pair)

  sc_agg = _build_sc_agg()
  g = _tc_pre(x, W1, dinv)
  part = sc_agg(g, src, dst, edge_weight)
  g = _tc_mid(part, g, dinv, b1, W2)
  part = sc_agg(g, src, dst, edge_weight)
  g = _tc_mid(part, g, dinv, b2, W3)
  part = sc_agg(g, src, dst, edge_weight)
  g = _tc_mid(part, g, dinv, b3, W4)
  part = sc_agg(g, src, dst, edge_weight)
  return _tc_fin(part, g, dinv, b4)


# speedup vs baseline: 21.1702x; 1.1988x over previous
import functools
import jax
import jax.numpy as jnp
from jax import lax
from jax.experimental import pallas as pl
from jax.experimental.pallas import tpu as pltpu
from jax.experimental.pallas import tpu_sc as plsc

N = 10000
E = 320000
D = 128

NC = 2
NS = 16
NW = NC * NS
EPT = E // NW
CHUNK = 80
NCHUNK = EPT // CHUNK
NP = 10240
RPT = NP // NS
ZR = RPT // 5
DEGP = 10240
DPT = DEGP // NS


@functools.cache
def _build_sc_degree():
  mesh = plsc.VectorSubcoreMesh(core_axis_name="c", subcore_axis_name="s")

  @functools.partial(
      pl.kernel,
      out_type=jax.ShapeDtypeStruct((NC, DEGP), jnp.float32),
      mesh=mesh,
      scratch_types=[
          pltpu.MemorySpace.VMEM_SHARED((DEGP,), jnp.float32),
          pltpu.MemorySpace.VMEM((DPT,), jnp.float32),
          [pltpu.MemorySpace.VMEM((CHUNK,), jnp.int32) for _ in range(8)],
          [pltpu.MemorySpace.VMEM((CHUNK,), jnp.float32) for _ in range(8)],
          [pltpu.SemaphoreType.DMA for _ in range(8)],
          [pltpu.SemaphoreType.DMA for _ in range(4)],
      ],
  )
  def sc_degree(dst_hbm, ew_hbm, deg_out, dacc, zv, dst_v, ew_v, isem, ssem):
    c = lax.axis_index("c")
    s = lax.axis_index("s")
    wid = c * NS + s
    base = wid * EPT

    def ifetch(j, k8):
      off = base + j * CHUNK
      pltpu.async_copy(dst_hbm.at[pl.ds(off, CHUNK)], dst_v[k8], isem[k8])
      pltpu.async_copy(ew_hbm.at[pl.ds(off, CHUNK)], ew_v[k8], isem[k8])

    def iwait(k8):
      pltpu.make_async_copy(dst_hbm.at[pl.ds(0, CHUNK)], dst_v[k8], isem[k8]).wait()
      pltpu.make_async_copy(ew_hbm.at[pl.ds(0, CHUNK)], ew_v[k8], isem[k8]).wait()

    def swait(k4):
      pltpu.make_async_copy(ew_v[0], dacc.at[dst_v[0]], ssem[k4]).wait()

    for k in range(6):
      ifetch(k, k)

    for j in range(DPT // 16):
      zv[pl.ds(j * 16, 16)] = jnp.zeros((16,), jnp.float32)
    pltpu.sync_copy(zv, dacc.at[pl.ds(s * DPT, DPT)])
    plsc.subcore_barrier()

    def emit(j, k4, k8, has_swait, has_fetch):
      iwait(k8)
      pltpu.async_copy(ew_v[k8], dacc.at[dst_v[k8]], ssem[k4], add=True)
      if has_swait:
        swait((k4 + 2) % 4)
      if has_fetch:
        ifetch(j + 6, (k8 + 6) % 8)

    for j in range(8):
      emit(j, j % 4, j % 8, j >= 2, True)

    def body(q, carry):
      j0 = 8 * q
      for k in range(8):
        emit(j0 + k, k % 4, k, True, True)
      return carry

    lax.fori_loop(1, 14, body, 0)
    for j in range(112, NCHUNK):
      emit(j, j % 4, j % 8, True, j + 6 < NCHUNK)
    swait((NCHUNK - 2) % 4)
    swait((NCHUNK - 1) % 4)

    plsc.subcore_barrier()
    pltpu.sync_copy(dacc.at[pl.ds(s * DPT, DPT)],
                    deg_out.at[c, pl.ds(s * DPT, DPT)])

  return sc_degree


@functools.cache
def _build_sc_agg():
  mesh = plsc.VectorSubcoreMesh(core_axis_name="c", subcore_axis_name="s")

  @functools.partial(
      pl.kernel,
      out_type=jax.ShapeDtypeStruct((NC, NP, D), jnp.float32),
      mesh=mesh,
      scratch_types=[
          pltpu.MemorySpace.VMEM_SHARED((NP, D), jnp.float32),
          [pltpu.MemorySpace.VMEM((CHUNK, D), jnp.float32) for _ in range(4)],
          [pltpu.MemorySpace.VMEM((CHUNK,), jnp.int32) for _ in range(8)],
          [pltpu.MemorySpace.VMEM((CHUNK,), jnp.int32) for _ in range(8)],
          [pltpu.MemorySpace.VMEM((CHUNK,), jnp.float32) for _ in range(8)],
          [pltpu.SemaphoreType.DMA for _ in range(4)],
          [pltpu.SemaphoreType.DMA for _ in range(4)],
          [pltpu.SemaphoreType.DMA for _ in range(8)],
      ],
  )
  def sc_agg(g_hbm, src_hbm, dst_hbm, ew_hbm, part_out,
             acc, rows, src_v, dst_v, ew_v, gsem, ssem, isem):
    c = lax.axis_index("c")
    s = lax.axis_index("s")
    wid = c * NS + s
    base = wid * EPT

    def ifetch(j, k8):
      off = base + j * CHUNK
      pltpu.async_copy(src_hbm.at[pl.ds(off, CHUNK)], src_v[k8], isem[k8])
      pltpu.async_copy(dst_hbm.at[pl.ds(off, CHUNK)], dst_v[k8], isem[k8])
      pltpu.async_copy(ew_hbm.at[pl.ds(off, CHUNK)], ew_v[k8], isem[k8])

    def iwait(k8):
      pltpu.make_async_copy(src_hbm.at[pl.ds(0, CHUNK)], src_v[k8], isem[k8]).wait()
      pltpu.make_async_copy(dst_hbm.at[pl.ds(0, CHUNK)], dst_v[k8], isem[k8]).wait()
      pltpu.make_async_copy(ew_hbm.at[pl.ds(0, CHUNK)], ew_v[k8], isem[k8]).wait()

    def gather(k8, k4):
      pltpu.async_copy(g_hbm.at[src_v[k8]], rows[k4], gsem[k4])

    def gwait(k4):
      pltpu.make_async_copy(g_hbm.at[src_v[0]], rows[k4], gsem[k4]).wait()

    def swait(k4):
      pltpu.make_async_copy(rows[0], acc.at[dst_v[0]], ssem[k4]).wait()

    for k in range(6):
      ifetch(k, k)

    def zrow(r, carry):
      for f in range(D // 16):
        rows[3][r, pl.ds(f * 16, 16)] = jnp.zeros((16,), jnp.float32)
      return carry

    lax.fori_loop(0, CHUNK, zrow, 0)
    for j in range(RPT // CHUNK):
      pltpu.sync_copy(rows[3], acc.at[pl.ds(s * RPT + j * CHUNK, CHUNK)])
    plsc.subcore_barrier()

    iwait(0)
    gather(0, 0)
    iwait(1)
    gather(1, 1)
    pltpu.async_copy(rows[3], acc.at[dst_v[0]], ssem[2], add=True)
    pltpu.async_copy(rows[3], acc.at[dst_v[0]], ssem[3], add=True)

    def scale(k4, k8):
      r = rows[k4]

      def scale_m(m, c2):
        ew16 = ew_v[k8][pl.ds(m * 16, 16)]
        for t in range(16):
          w = ew16[t]
          e = m * 16 + t
          for f in range(D // 16):
            r[e, pl.ds(f * 16, 16)] = r[e, pl.ds(f * 16, 16)] * w
        return c2

      lax.fori_loop(0, CHUNK // 16, scale_m, 0)

    def emit(j, k4, k8):
      gwait(k4)
      scale(k4, k8)
      pltpu.async_copy(rows[k4], acc.at[dst_v[k8]], ssem[k4], add=True)
      swait((k4 + 2) % 4)
      ifetch(jnp.minimum(j + 6, NCHUNK - 1), (k8 + 6) % 8)
      iwait((k8 + 2) % 8)
      gather((k8 + 2) % 8, (k4 + 2) % 4)

    def body(q, carry):
      j0 = 8 * q
      for k in range(8):
        emit(j0 + k, k % 4, k)
      return carry

    lax.fori_loop(0, 15, body, 0)
    for j in range(120, NCHUNK):
      emit(j, j % 4, j % 8)
    gwait(1)
    gwait(2)
    swait((NCHUNK - 2) % 4)
    swait((NCHUNK - 1) % 4)
    for st in (7, 0, 1, 2):
      iwait(st)

    plsc.subcore_barrier()
    pltpu.sync_copy(acc.at[pl.ds(s * RPT, RPT)],
                    part_out.at[c, pl.ds(s * RPT, RPT)])

  return sc_agg


def _dinv_body(dp_ref, out_ref):
  deg = dp_ref[0] + dp_ref[1] + 1.0
  out_ref[...] = jnp.where(deg > 0, 1.0 / jnp.sqrt(deg), 0.0)


def _tc_dinv(deg_pair):
  dp = deg_pair.reshape(NC, DEGP // D, D)
  out = pl.pallas_call(
      _dinv_body,
      out_shape=jax.ShapeDtypeStruct((DEGP // D, D), jnp.float32),
  )(dp)
  return out.reshape(DEGP, 1)[:N]


BLK = 1000
GRID = N // BLK


def _pre_body(x_ref, w_ref, dinv_ref, g_ref):
  h = jnp.dot(x_ref[...], w_ref[...], preferred_element_type=jnp.float32)
  g_ref[...] = h * dinv_ref[...]


def _tc_pre(x, W, dinv):
  return pl.pallas_call(
      _pre_body,
      grid=(GRID,),
      in_specs=[
          pl.BlockSpec((BLK, D), lambda i: (i, 0)),
          pl.BlockSpec((D, D), lambda i: (0, 0)),
          pl.BlockSpec((BLK, 1), lambda i: (i, 0)),
      ],
      out_specs=pl.BlockSpec((BLK, D), lambda i: (i, 0)),
      out_shape=jax.ShapeDtypeStruct((N, D), jnp.float32),
  )(x, W, dinv)


def _mid_body(p0_ref, p1_ref, g_ref, dinv_ref, b_ref, w_ref, out_ref):
  a = jax.nn.relu(dinv_ref[...] * (p0_ref[...] + p1_ref[...] + g_ref[...])
                  + b_ref[...])
  h = jnp.dot(a, w_ref[...], preferred_element_type=jnp.float32)
  out_ref[...] = h * dinv_ref[...]


def _tc_mid(part, g, dinv, b, Wn):
  return pl.pallas_call(
      _mid_body,
      grid=(GRID,),
      in_specs=[
          pl.BlockSpec((BLK, D), lambda i: (i, 0)),
          pl.BlockSpec((BLK, D), lambda i: (i, 0)),
          pl.BlockSpec((BLK, D), lambda i: (i, 0)),
          pl.BlockSpec((BLK, 1), lambda i: (i, 0)),
          pl.BlockSpec((1, D), lambda i: (0, 0)),
          pl.BlockSpec((D, D), lambda i: (0, 0)),
      ],
      out_specs=pl.BlockSpec((BLK, D), lambda i: (i, 0)),
      out_shape=jax.ShapeDtypeStruct((N, D), jnp.float32),
  )(part[0], part[1], g, dinv, b.reshape(1, D), Wn)


def _fin_body(p0_ref, p1_ref, g_ref, dinv_ref, b_ref, out_ref):
  out_ref[...] = jax.nn.sigmoid(
      dinv_ref[...] * (p0_ref[...] + p1_ref[...] + g_ref[...]) + b_ref[...])


def _tc_fin(part, g, dinv, b):
  return pl.pallas_call(
      _fin_body,
      grid=(GRID,),
      in_specs=[
          pl.BlockSpec((BLK, D), lambda i: (i, 0)),
          pl.BlockSpec((BLK, D), lambda i: (i, 0)),
          pl.BlockSpec((BLK, D), lambda i: (i, 0)),
          pl.BlockSpec((BLK, 1), lambda i: (i, 0)),
          pl.BlockSpec((1, D), lambda i: (0, 0)),
      ],
      out_specs=pl.BlockSpec((BLK, D), lambda i: (i, 0)),
      out_shape=jax.ShapeDtypeStruct((N, D), jnp.float32),
  )(part[0], part[1], g, dinv, b.reshape(1, D))


@jax.jit
def kernel(x, edge_index, edge_weight, W1, b1, W2, b2, W3, b3, W4, b4):
  src = edge_index[0]
  dst = edge_index[1]


  deg_pair = _build_sc_degree()(dst, edge_weight)
  dinv = _tc_dinv(deg_pair)

  sc_agg = _build_sc_agg()
  g = _tc_pre(x, W1, dinv)
  part = sc_agg(g, src, dst, edge_weight)
  g = _tc_mid(part, g, dinv, b1, W2)
  part = sc_agg(g, src, dst, edge_weight)
  g = _tc_mid(part, g, dinv, b2, W3)
  part = sc_agg(g, src, dst, edge_weight)
  g = _tc_mid(part, g, dinv, b3, W4)
  part = sc_agg(g, src, dst, edge_weight)
  return _tc_fin(part, g, dinv, b4)
